# baseline (speedup 1.0000x reference)
import jax
import jax.numpy as jnp
from jax import lax
from jax.experimental import pallas as pl
from jax.experimental.pallas import tpu as pltpu
from jax.experimental.pallas import tpu_sc as plsc

B = 512
NV = 4096
VB = 256
RT = 2048
L = 16
NC = 2
NS = 16
NW = NC * NS
RS = (NV - RT) // NW
CR = 16
LIM = 0.125


def _tc_body(ts_ref, off_ref, ph_ref, th_ref, out_ref, acc_ref):
    i = pl.program_id(0)

    @pl.when(i == 0)
    def _init():
        acc_ref[...] = jnp.zeros_like(acc_ref)

    rr = lax.broadcasted_iota(jnp.int32, (VB, 1), 0)
    b1 = (rr >> 4).astype(jnp.float32) * 0.125 - 1.0
    b2 = (rr & 15).astype(jnp.float32) * 0.125 - 1.0

    t0 = jnp.clip(ts_ref[0:1, :] - (i.astype(jnp.float32) * 0.125 - 1.0),
                  -LIM, LIM)
    t1 = jnp.clip(ts_ref[1:2, :] - b1, -LIM, LIM)
    t2 = jnp.clip(ts_ref[2:3, :] - b2, -LIM, LIM)

    m = jnp.logical_and(ph_ref[1] > ph_ref[0], th_ref[...] >= 0.5)
    mf = m.astype(jnp.float32)
    s = (jnp.abs(off_ref[0] - t0) + jnp.abs(off_ref[1] - t1)
         + jnp.abs(off_ref[2] - t2))
    acc_ref[0:1, :] += jnp.sum(s * mf, axis=0, keepdims=True)
    acc_ref[1:2, :] += jnp.sum(mf, axis=0, keepdims=True)

    @pl.when(i == RT // VB - 1)
    def _finish():
        out_ref[0, 0] = jnp.sum(acc_ref[0:1, :])
        out_ref[1, 0] = jnp.sum(acc_ref[1:2, :])


def _tc_partials(ts_t, off_t, ph_t, th_t):
    return pl.pallas_call(
        _tc_body,
        grid=(RT // VB,),
        in_specs=[
            pl.BlockSpec((3, B), lambda i: (0, 0)),
            pl.BlockSpec((3, VB, B), lambda i: (0, i, 0)),
            pl.BlockSpec((2, VB, B), lambda i: (0, i, 0)),
            pl.BlockSpec((VB, B), lambda i: (i, 0)),
        ],
        out_specs=pl.BlockSpec(memory_space=pltpu.SMEM),
        out_shape=jax.ShapeDtypeStruct((2, 1), jnp.float32),
        scratch_shapes=[pltpu.VMEM((2, B), jnp.float32)],
    )(ts_t, off_t, ph_t, th_t)


def _sc_start(off_t, ph_t, th_t, r0, off_buf, ph_buf, th_buf, sem):
    cps = [pltpu.async_copy(off_t.at[c, pl.ds(r0, CR), :], off_buf.at[c], sem)
           for c in range(3)]
    cps += [pltpu.async_copy(ph_t.at[c, pl.ds(r0, CR), :], ph_buf.at[c], sem)
            for c in range(2)]
    cps.append(pltpu.async_copy(th_t.at[pl.ds(r0, CR), :], th_buf, sem))
    return cps


def _sc_chunk(off_buf, ph_buf, th_buf, ts_buf, r0, accs):
    b0 = (r0 >> 8).astype(jnp.float32) * 0.125 - 1.0
    b1 = ((r0 >> 4) & 15).astype(jnp.float32) * 0.125 - 1.0

    def group(g, accs, b0=b0, b1=b1):
        a = list(accs)
        gl = g * L
        t0 = jnp.clip(ts_buf[0, pl.ds(gl, L)] - b0, -LIM, LIM)
        t1 = jnp.clip(ts_buf[1, pl.ds(gl, L)] - b1, -LIM, LIM)
        ts2 = ts_buf[2, pl.ds(gl, L)]
        for r in range(CR):
            t2 = jnp.clip(ts2 - (r * 0.125 - 1.0), -LIM, LIM)
            o0 = off_buf[0, r, pl.ds(gl, L)]
            o1 = off_buf[1, r, pl.ds(gl, L)]
            o2 = off_buf[2, r, pl.ds(gl, L)]
            p0 = ph_buf[0, r, pl.ds(gl, L)]
            p1 = ph_buf[1, r, pl.ds(gl, L)]
            tt = th_buf[r, pl.ds(gl, L)]
            m = jnp.logical_and(p1 > p0, tt >= 0.5)
            s = jnp.abs(o0 - t0) + jnp.abs(o1 - t1) + jnp.abs(o2 - t2)
            k = r % 4
            a[k] = a[k] + jnp.where(m, s, 0.0)
            a[4 + k] = a[4 + k] + jnp.where(m, 1.0, 0.0)
        return tuple(a)

    return plsc.parallel_loop(0, B // L, 1, carry=accs)(group)


def _sc_body(off_t, ph_t, th_t, ts_t, out_hbm,
             off0, ph0, th0, off1, ph1, th1, ts_buf, res_buf, sem0, sem1):
    wid = lax.axis_index("s") * NC + lax.axis_index("c")
    base = RT + wid * RS
    pltpu.sync_copy(ts_t, ts_buf)

    zero = jnp.zeros((L,), jnp.float32)
    accs = (zero,) * 8

    nch = RS // CR
    slots = ((off0, ph0, th0, sem0), (off1, ph1, th1, sem1))
    pending = [None, None]
    pending[0] = _sc_start(off_t, ph_t, th_t, base, *slots[0])
    for i in range(nch):
        s = i % 2
        if i + 1 < nch:
            pending[1 - s] = _sc_start(off_t, ph_t, th_t,
                                       base + (i + 1) * CR, *slots[1 - s])
        for cp in pending[s]:
            cp.wait()
        obuf, pbuf, tbuf, _ = slots[s]
        accs = _sc_chunk(obuf, pbuf, tbuf, ts_buf, base + i * CR, accs)

    tot = (accs[0] + accs[1]) + (accs[2] + accs[3])
    cnt = (accs[4] + accs[5]) + (accs[6] + accs[7])
    res_buf[0, :] = tot
    res_buf[1, :] = cnt
    pltpu.sync_copy(res_buf, out_hbm.at[wid])


def _sc_partials(off_t, ph_t, th_t, ts_t):
    mesh = plsc.VectorSubcoreMesh(core_axis_name="c", subcore_axis_name="s")
    f = pl.kernel(
        _sc_body,
        out_type=jax.ShapeDtypeStruct((NW, 2, L), jnp.float32),
        mesh=mesh,
        scratch_types=[
            pltpu.VMEM((3, CR, B), jnp.float32),
            pltpu.VMEM((2, CR, B), jnp.float32),
            pltpu.VMEM((CR, B), jnp.float32),
            pltpu.VMEM((3, CR, B), jnp.float32),
            pltpu.VMEM((2, CR, B), jnp.float32),
            pltpu.VMEM((CR, B), jnp.float32),
            pltpu.VMEM((3, B), jnp.float32),
            pltpu.VMEM((2, L), jnp.float32),
            pltpu.SemaphoreType.DMA,
            pltpu.SemaphoreType.DMA,
        ],
    )
    return f(off_t, ph_t, th_t, ts_t)


def kernel(offsets, target_skeleton, predicted_heatmap, target_heatmap):
    off_t = jnp.transpose(offsets, (1, 2, 3, 4, 0)).reshape(3, NV, B)
    ph_t = jnp.transpose(predicted_heatmap, (1, 2, 3, 4, 0)).reshape(2, NV, B)
    th_t = jnp.transpose(target_heatmap, (1, 2, 3, 4, 0)).reshape(NV, B)
    ts_t = jnp.transpose(target_skeleton, (2, 1, 0)).reshape(3, B)

    tc = _tc_partials(ts_t, off_t, ph_t, th_t)
    sc = _sc_partials(off_t, ph_t, th_t, ts_t)
    tot = tc[0, 0] + jnp.sum(sc[:, 0, :])
    cnt = tc[1, 0] + jnp.sum(sc[:, 1, :])
    denom = jnp.maximum(cnt * 3.0, 1.0)
    return jnp.where(cnt > 0, tot / denom, 0.0)


# speedup vs baseline: 2.2499x; 2.2499x over previous
import jax
import jax.numpy as jnp
from jax import lax
from jax.experimental import pallas as pl
from jax.experimental.pallas import tpu as pltpu

B = 512
NV = 4096
VB = 512
GRID = NV // VB
LIM = 0.125


def _tc_body(ts_ref, off_ref, ph_ref, th_ref, out_ref, acc_ref):
    i = pl.program_id(0)

    @pl.when(i == 0)
    def _init():
        acc_ref[...] = jnp.zeros_like(acc_ref)

    rr = lax.broadcasted_iota(jnp.int32, (VB, 1), 0) + i * VB
    b0 = (rr >> 8).astype(jnp.float32) * 0.125 - 1.0
    b1 = ((rr >> 4) & 15).astype(jnp.float32) * 0.125 - 1.0
    b2 = (rr & 15).astype(jnp.float32) * 0.125 - 1.0

    t0 = jnp.clip(ts_ref[0:1, :] - b0, -LIM, LIM)
    t1 = jnp.clip(ts_ref[1:2, :] - b1, -LIM, LIM)
    t2 = jnp.clip(ts_ref[2:3, :] - b2, -LIM, LIM)

    m = jnp.logical_and(ph_ref[1] > ph_ref[0], th_ref[...] >= 0.5)
    mf = m.astype(jnp.float32)
    s = (jnp.abs(off_ref[0] - t0) + jnp.abs(off_ref[1] - t1)
         + jnp.abs(off_ref[2] - t2))
    acc_ref[0:1, :] += jnp.sum(s * mf, axis=0, keepdims=True)
    acc_ref[1:2, :] += jnp.sum(mf, axis=0, keepdims=True)

    @pl.when(i == GRID - 1)
    def _finish():
        tot = jnp.sum(acc_ref[0:1, :])
        cnt = jnp.sum(acc_ref[1:2, :])
        denom = jnp.maximum(cnt * 3.0, 1.0)
        out_ref[0, 0] = jnp.where(cnt > 0, tot / denom, 0.0)


def kernel(offsets, target_skeleton, predicted_heatmap, target_heatmap):
    off_t = jnp.transpose(offsets, (1, 2, 3, 4, 0)).reshape(3, NV, B)
    ph_t = jnp.transpose(predicted_heatmap, (1, 2, 3, 4, 0)).reshape(2, NV, B)
    th_t = jnp.transpose(target_heatmap, (1, 2, 3, 4, 0)).reshape(NV, B)
    ts_t = jnp.transpose(target_skeleton, (2, 1, 0)).reshape(3, B)

    out = pl.pallas_call(
        _tc_body,
        grid=(GRID,),
        in_specs=[
            pl.BlockSpec((3, B), lambda i: (0, 0)),
            pl.BlockSpec((3, VB, B), lambda i: (0, i, 0)),
            pl.BlockSpec((2, VB, B), lambda i: (0, i, 0)),
            pl.BlockSpec((VB, B), lambda i: (i, 0)),
        ],
        out_specs=pl.BlockSpec(memory_space=pltpu.SMEM),
        out_shape=jax.ShapeDtypeStruct((1, 1), jnp.float32),
        scratch_shapes=[pltpu.VMEM((2, B), jnp.float32)],
    )(ts_t, off_t, ph_t, th_t)
    return out[0, 0]
